# baseline (speedup 1.0000x reference)
import functools

import jax
import jax.numpy as jnp
import numpy as np
from jax import lax
from jax.experimental import pallas as pl
from jax.experimental.pallas import tpu as pltpu
from jax.experimental.pallas import tpu_sc as plsc

_N = 10000
_D = 128
_NPAD = 10240
_CHUNK = 128
_NW = 32
_ROWS_PER_TILE = 624
_TAIL_BASE = 16 * _ROWS_PER_TILE
_TAIL = _N - _TAIL_BASE
_ZCHUNKS = _NPAD // (16 * _CHUNK)


def _copy_out(acc, out_hbm, c, s):
    base = s * _ROWS_PER_TILE
    pltpu.sync_copy(
        acc.at[pl.ds(base, _ROWS_PER_TILE)],
        out_hbm.at[c].at[pl.ds(base, _ROWS_PER_TILE)],
    )

    @pl.when(s == 15)
    def _():
        pltpu.sync_copy(
            acc.at[pl.ds(_TAIL_BASE, _TAIL)],
            out_hbm.at[c].at[pl.ds(_TAIL_BASE, _TAIL)],
        )

_NBUF = 2
_IBUF = 4


def _make_propagate(nchunk):
    assert nchunk % _IBUF == 0

    @functools.partial(
        pl.kernel,
        out_type=jax.ShapeDtypeStruct((2, _N, _D), jnp.float32),
        mesh=plsc.VectorSubcoreMesh(core_axis_name="c", subcore_axis_name="s"),
        scratch_types=[
            pltpu.VMEM((_IBUF, _CHUNK), jnp.int32),
            pltpu.VMEM((nchunk, _CHUNK), jnp.int32),
            [pltpu.VMEM((_CHUNK, _D), jnp.float32) for _ in range(_NBUF)],
            [pltpu.SemaphoreType.DMA for _ in range(_NBUF)],
            [pltpu.SemaphoreType.DMA for _ in range(_IBUF)],
            [pltpu.SemaphoreType.DMA for _ in range(_NBUF)],
            pltpu.VMEM_SHARED((_NPAD, _D), jnp.float32),
        ],
    )
    def propagate(
        g_hbm, srcs_hbm, dsts_hbm, out_hbm, sidx, dst_v, rows, gsems, isems, ssems, acc
    ):
        c = lax.axis_index("c")
        s = lax.axis_index("s")
        wid = s * 2 + c
        pltpu.sync_copy(dsts_hbm.at[wid], dst_v)

        def zrow(r, carry):
            for k in range(_D // 16):
                rows[0][r, pl.ds(k * 16, 16)] = jnp.zeros((16,), jnp.float32)
            return carry

        lax.fori_loop(0, _CHUNK, zrow, 0)
        for t in range(_ZCHUNKS):
            pltpu.sync_copy(rows[0], acc.at[pl.ds((s * _ZCHUNKS + t) * _CHUNK, _CHUNK)])
        plsc.subcore_barrier()

        for i in range(_IBUF):
            pltpu.async_copy(srcs_hbm.at[wid].at[i], sidx.at[i], isems[i])
        for b in range(_NBUF):
            pltpu.make_async_copy(srcs_hbm.at[wid].at[b], sidx.at[b], isems[b]).wait()
            pltpu.async_copy(g_hbm.at[sidx.at[b]], rows[b], gsems[b])

        def group(p, carry):
            j = p * _IBUF
            for b in range(_IBUF):
                k = j + b
                rb = b % _NBUF
                pltpu.make_async_copy(g_hbm.at[sidx.at[b]], rows[rb], gsems[rb]).wait()
                pltpu.async_copy(rows[rb], acc.at[dst_v.at[k]], ssems[rb], add=True)

                sl = (b + _NBUF) % _IBUF

                @pl.when(k + _NBUF < nchunk)
                def _():
                    pltpu.make_async_copy(
                        srcs_hbm.at[wid].at[k + _NBUF], sidx.at[sl], isems[sl]
                    ).wait()
                    pltpu.make_async_copy(
                        rows[rb], acc.at[dst_v.at[k]], ssems[rb]
                    ).wait()
                    pltpu.async_copy(g_hbm.at[sidx.at[sl]], rows[rb], gsems[rb])

                @pl.when(k + _IBUF < nchunk)
                def _():
                    pltpu.async_copy(
                        srcs_hbm.at[wid].at[k + _IBUF], sidx.at[b], isems[b]
                    )

            return carry

        lax.fori_loop(0, nchunk // _IBUF, group, 0)
        for b in range(_NBUF):
            pltpu.make_async_copy(
                rows[b], acc.at[dst_v.at[nchunk - _NBUF + b]], ssems[b]
            ).wait()
        plsc.subcore_barrier()
        _copy_out(acc, out_hbm, c, s)

    return propagate


def _make_degree(nchunk):
    _SBUF = 8
    assert nchunk % _SBUF == 0

    @functools.partial(
        pl.kernel,
        out_type=jax.ShapeDtypeStruct((2, _N, _D), jnp.float32),
        mesh=plsc.VectorSubcoreMesh(core_axis_name="c", subcore_axis_name="s"),
        scratch_types=[
            pltpu.VMEM((nchunk, _CHUNK), jnp.int32),
            pltpu.VMEM((_CHUNK, _D), jnp.float32),
            [pltpu.SemaphoreType.DMA for _ in range(_SBUF)],
            pltpu.VMEM_SHARED((_NPAD, _D), jnp.float32),
        ],
    )
    def degree(dsts_hbm, out_hbm, dst_v, ones_v, sems, acc):
        c = lax.axis_index("c")
        s = lax.axis_index("s")
        wid = s * 2 + c
        pltpu.sync_copy(dsts_hbm.at[wid], dst_v)

        def fill(val):
            def row(r, carry):
                for k in range(_D // 16):
                    ones_v[r, pl.ds(k * 16, 16)] = jnp.full((16,), val, jnp.float32)
                return carry

            lax.fori_loop(0, _CHUNK, row, 0)

        fill(0.0)
        for t in range(_ZCHUNKS):
            pltpu.sync_copy(ones_v, acc.at[pl.ds((s * _ZCHUNKS + t) * _CHUNK, _CHUNK)])
        plsc.subcore_barrier()
        fill(1.0)

        for b in range(_SBUF):
            pltpu.async_copy(ones_v, acc.at[dst_v.at[b]], sems[b], add=True)

        def group(p, carry):
            j = p * _SBUF
            for b in range(_SBUF):
                k = j + b
                pltpu.make_async_copy(ones_v, acc.at[dst_v.at[k]], sems[b]).wait()

                @pl.when(k + _SBUF < nchunk)
                def _():
                    pltpu.async_copy(
                        ones_v, acc.at[dst_v.at[k + _SBUF]], sems[b], add=True
                    )

            return carry

        lax.fori_loop(0, nchunk // _SBUF, group, 0)
        plsc.subcore_barrier()
        _copy_out(acc, out_hbm, c, s)

    return degree


_BLK = 2000
_BN_RSQRT = float(1.0 / np.sqrt(1.0 + 1e-5))


def _mm_body(x_ref, w_ref, z_ref):
    z_ref[...] = jnp.dot(x_ref[...], w_ref[...], preferred_element_type=jnp.float32)


_mm = pl.pallas_call(
    _mm_body,
    grid=(_N // _BLK,),
    in_specs=[
        pl.BlockSpec((_BLK, _D), lambda i: (i, 0)),
        pl.BlockSpec((_D, _D), lambda i: (0, 0)),
    ],
    out_specs=pl.BlockSpec((_BLK, _D), lambda i: (i, 0)),
    out_shape=jax.ShapeDtypeStruct((_N, _D), jnp.float32),
)


def _pre_body(z_ref, deg_ref, g_ref, dinv_ref):
    deg = deg_ref[0, :, 0:1] + deg_ref[1, :, 0:1] + 1.0
    dinv = lax.rsqrt(deg)
    dinv_ref[...] = dinv
    g_ref[...] = dinv * z_ref[...]


_pre = pl.pallas_call(
    _pre_body,
    grid=(_N // _BLK,),
    in_specs=[
        pl.BlockSpec((_BLK, _D), lambda i: (i, 0)),
        pl.BlockSpec((2, _BLK, _D), lambda i: (0, i, 0)),
    ],
    out_specs=[
        pl.BlockSpec((_BLK, _D), lambda i: (i, 0)),
        pl.BlockSpec((_BLK, 1), lambda i: (i, 0)),
    ],
    out_shape=[
        jax.ShapeDtypeStruct((_N, _D), jnp.float32),
        jax.ShapeDtypeStruct((_N, 1), jnp.float32),
    ],
)


def _mid_body(s_ref, g_ref, dinv_ref, w_ref, b_ref, gam_ref, bet_ref, out_ref):
    acc = s_ref[0] + s_ref[1] + g_ref[...]
    conv = dinv_ref[...] * acc + b_ref[...]
    h = jnp.maximum(conv * (gam_ref[...] * _BN_RSQRT) + bet_ref[...], 0.0)
    out_ref[...] = dinv_ref[...] * jnp.dot(
        h, w_ref[...], preferred_element_type=jnp.float32
    )


_mid = pl.pallas_call(
    _mid_body,
    grid=(_N // _BLK,),
    in_specs=[
        pl.BlockSpec((2, _BLK, _D), lambda i: (0, i, 0)),
        pl.BlockSpec((_BLK, _D), lambda i: (i, 0)),
        pl.BlockSpec((_BLK, 1), lambda i: (i, 0)),
        pl.BlockSpec((_D, _D), lambda i: (0, 0)),
        pl.BlockSpec((1, _D), lambda i: (0, 0)),
        pl.BlockSpec((1, _D), lambda i: (0, 0)),
        pl.BlockSpec((1, _D), lambda i: (0, 0)),
    ],
    out_specs=pl.BlockSpec((_BLK, _D), lambda i: (i, 0)),
    out_shape=jax.ShapeDtypeStruct((_N, _D), jnp.float32),
)


def _fin_body(s_ref, g_ref, dinv_ref, b_ref, out_ref):
    conv = dinv_ref[...] * (s_ref[0] + s_ref[1] + g_ref[...]) + b_ref[...]
    m = jnp.max(conv, axis=-1, keepdims=True)
    lse = jnp.log(jnp.sum(jnp.exp(conv - m), axis=-1, keepdims=True)) + m
    out_ref[...] = conv - lse


_fin = pl.pallas_call(
    _fin_body,
    grid=(_N // _BLK,),
    in_specs=[
        pl.BlockSpec((2, _BLK, _D), lambda i: (0, i, 0)),
        pl.BlockSpec((_BLK, _D), lambda i: (i, 0)),
        pl.BlockSpec((_BLK, 1), lambda i: (i, 0)),
        pl.BlockSpec((1, _D), lambda i: (0, 0)),
    ],
    out_specs=pl.BlockSpec((_BLK, _D), lambda i: (i, 0)),
    out_shape=jax.ShapeDtypeStruct((_N, _D), jnp.float32),
)


def kernel(x, edge_index, W, b, bn_gamma, bn_beta):
    e = edge_index.shape[1]
    nchunk = -(-e // (_NW * _CHUNK))
    nchunk = -(-nchunk // 8) * 8
    epad = _NW * nchunk * _CHUNK
    pad = epad - e
    pad_ar = jnp.arange(pad, dtype=jnp.int32)
    src = jnp.concatenate([edge_index[0], pad_ar % _N])
    dst = jnp.concatenate([edge_index[1], _N + pad_ar % (_NPAD - _N)])
    src = src.reshape(nchunk, _NW, _CHUNK).swapaxes(0, 1)
    dst = dst.reshape(nchunk, _NW, _CHUNK).swapaxes(0, 1)

    propagate = _make_propagate(nchunk)
    z = _mm(x, W[0])
    deg = _make_degree(nchunk)(dst)

    g, dinv = _pre(z, deg)
    for i in range(4):
        s = propagate(g, src, dst)
        g = _mid(
            s,
            g,
            dinv,
            W[i + 1],
            b[i].reshape(1, _D),
            bn_gamma[i].reshape(1, _D),
            bn_beta[i].reshape(1, _D),
        )
    s = propagate(g, src, dst)
    return _fin(s, g, dinv, b[4].reshape(1, _D))


# speedup vs baseline: 1.0400x; 1.0400x over previous
import functools

import jax
import jax.numpy as jnp
import numpy as np
from jax import lax
from jax.experimental import pallas as pl
from jax.experimental.pallas import tpu as pltpu
from jax.experimental.pallas import tpu_sc as plsc

_N = 10000
_D = 128
_NPAD = 10240
_CHUNK = 128
_NW = 32
_ROWS_PER_TILE = 624
_TAIL_BASE = 16 * _ROWS_PER_TILE
_TAIL = _N - _TAIL_BASE
_ZCHUNKS = _NPAD // (16 * _CHUNK)


def _copy_out(acc, out_hbm, c, s):
    base = s * _ROWS_PER_TILE
    pltpu.sync_copy(
        acc.at[pl.ds(base, _ROWS_PER_TILE)],
        out_hbm.at[c].at[pl.ds(base, _ROWS_PER_TILE)],
    )

    @pl.when(s == 15)
    def _():
        pltpu.sync_copy(
            acc.at[pl.ds(_TAIL_BASE, _TAIL)],
            out_hbm.at[c].at[pl.ds(_TAIL_BASE, _TAIL)],
        )

_NBUF = 2
_IBUF = 4


def _make_propagate(nchunk):
    assert nchunk % _IBUF == 0

    @functools.partial(
        pl.kernel,
        out_type=jax.ShapeDtypeStruct((2, _N, _D), jnp.float32),
        mesh=plsc.VectorSubcoreMesh(core_axis_name="c", subcore_axis_name="s"),
        scratch_types=[
            pltpu.VMEM((_IBUF, _CHUNK), jnp.int32),
            pltpu.VMEM((nchunk, _CHUNK), jnp.int32),
            [pltpu.VMEM((_CHUNK, _D), jnp.float32) for _ in range(_NBUF)],
            [pltpu.SemaphoreType.DMA for _ in range(_NBUF)],
            [pltpu.SemaphoreType.DMA for _ in range(_IBUF)],
            [pltpu.SemaphoreType.DMA for _ in range(_NBUF)],
            pltpu.VMEM_SHARED((_NPAD, _D), jnp.float32),
        ],
    )
    def propagate(
        g_hbm, srcs_hbm, dsts_hbm, out_hbm, sidx, dst_v, rows, gsems, isems, ssems, acc
    ):
        c = lax.axis_index("c")
        s = lax.axis_index("s")
        wid = s * 2 + c
        pltpu.sync_copy(dsts_hbm.at[wid], dst_v)

        def zrow(r, carry):
            for k in range(_D // 16):
                rows[0][r, pl.ds(k * 16, 16)] = jnp.zeros((16,), jnp.float32)
            return carry

        lax.fori_loop(0, _CHUNK, zrow, 0)
        for t in range(_ZCHUNKS):
            pltpu.sync_copy(rows[0], acc.at[pl.ds((s * _ZCHUNKS + t) * _CHUNK, _CHUNK)])
        plsc.subcore_barrier()

        for i in range(_IBUF):
            pltpu.async_copy(srcs_hbm.at[wid].at[i], sidx.at[i], isems[i])
        for b in range(_NBUF):
            pltpu.make_async_copy(srcs_hbm.at[wid].at[b], sidx.at[b], isems[b]).wait()
            pltpu.async_copy(g_hbm.at[sidx.at[b]], rows[b], gsems[b])

        def group(p, carry):
            j = p * _IBUF
            for b in range(_IBUF):
                k = j + b
                rb = b % _NBUF
                pltpu.make_async_copy(g_hbm.at[sidx.at[b]], rows[rb], gsems[rb]).wait()
                pltpu.async_copy(rows[rb], acc.at[dst_v.at[k]], ssems[rb], add=True)

                sl = (b + _NBUF) % _IBUF

                @pl.when(k + _NBUF < nchunk)
                def _():
                    pltpu.make_async_copy(
                        srcs_hbm.at[wid].at[k + _NBUF], sidx.at[sl], isems[sl]
                    ).wait()
                    pltpu.make_async_copy(
                        rows[rb], acc.at[dst_v.at[k]], ssems[rb]
                    ).wait()
                    pltpu.async_copy(g_hbm.at[sidx.at[sl]], rows[rb], gsems[rb])

                @pl.when(k + _IBUF < nchunk)
                def _():
                    pltpu.async_copy(
                        srcs_hbm.at[wid].at[k + _IBUF], sidx.at[b], isems[b]
                    )

            return carry

        lax.fori_loop(0, nchunk // _IBUF, group, 0)
        for b in range(_NBUF):
            pltpu.make_async_copy(
                rows[b], acc.at[dst_v.at[nchunk - _NBUF + b]], ssems[b]
            ).wait()
        plsc.subcore_barrier()
        _copy_out(acc, out_hbm, c, s)

    return propagate


_DW = 64


def _make_degree(nchunk):
    _SBUF = 8
    assert nchunk % _SBUF == 0

    @functools.partial(
        pl.kernel,
        out_type=jax.ShapeDtypeStruct((2, _N, _DW), jnp.float32),
        mesh=plsc.VectorSubcoreMesh(core_axis_name="c", subcore_axis_name="s"),
        scratch_types=[
            pltpu.VMEM((nchunk, _CHUNK), jnp.int32),
            pltpu.VMEM((_CHUNK, _DW), jnp.float32),
            [pltpu.SemaphoreType.DMA for _ in range(_SBUF)],
            pltpu.VMEM_SHARED((_NPAD, _DW), jnp.float32),
        ],
    )
    def degree(dsts_hbm, out_hbm, dst_v, ones_v, sems, acc):
        c = lax.axis_index("c")
        s = lax.axis_index("s")
        wid = s * 2 + c
        pltpu.sync_copy(dsts_hbm.at[wid], dst_v)

        def fill(val):
            def row(r, carry):
                for k in range(_DW // 16):
                    ones_v[r, pl.ds(k * 16, 16)] = jnp.full((16,), val, jnp.float32)
                return carry

            lax.fori_loop(0, _CHUNK, row, 0)

        fill(0.0)
        for t in range(_ZCHUNKS):
            pltpu.sync_copy(ones_v, acc.at[pl.ds((s * _ZCHUNKS + t) * _CHUNK, _CHUNK)])
        plsc.subcore_barrier()
        fill(1.0)

        for b in range(_SBUF):
            pltpu.async_copy(ones_v, acc.at[dst_v.at[b]], sems[b], add=True)

        def group(p, carry):
            j = p * _SBUF
            for b in range(_SBUF):
                k = j + b
                pltpu.make_async_copy(ones_v, acc.at[dst_v.at[k]], sems[b]).wait()

                @pl.when(k + _SBUF < nchunk)
                def _():
                    pltpu.async_copy(
                        ones_v, acc.at[dst_v.at[k + _SBUF]], sems[b], add=True
                    )

            return carry

        lax.fori_loop(0, nchunk // _SBUF, group, 0)
        plsc.subcore_barrier()
        _copy_out(acc, out_hbm, c, s)

    return degree


_BLK = 2000
_BN_RSQRT = float(1.0 / np.sqrt(1.0 + 1e-5))


def _mm_body(x_ref, w_ref, z_ref):
    z_ref[...] = jnp.dot(x_ref[...], w_ref[...], preferred_element_type=jnp.float32)


_mm = pl.pallas_call(
    _mm_body,
    grid=(_N // _BLK,),
    in_specs=[
        pl.BlockSpec((_BLK, _D), lambda i: (i, 0)),
        pl.BlockSpec((_D, _D), lambda i: (0, 0)),
    ],
    out_specs=pl.BlockSpec((_BLK, _D), lambda i: (i, 0)),
    out_shape=jax.ShapeDtypeStruct((_N, _D), jnp.float32),
)


def _pre_body(z_ref, deg_ref, g_ref, dinv_ref):
    deg = deg_ref[0, :, 0:1] + deg_ref[1, :, 0:1] + 1.0
    dinv = lax.rsqrt(deg)
    dinv_ref[...] = dinv
    g_ref[...] = dinv * z_ref[...]


_pre = pl.pallas_call(
    _pre_body,
    grid=(_N // _BLK,),
    in_specs=[
        pl.BlockSpec((_BLK, _D), lambda i: (i, 0)),
        pl.BlockSpec((2, _BLK, _DW), lambda i: (0, i, 0)),
    ],
    out_specs=[
        pl.BlockSpec((_BLK, _D), lambda i: (i, 0)),
        pl.BlockSpec((_BLK, 1), lambda i: (i, 0)),
    ],
    out_shape=[
        jax.ShapeDtypeStruct((_N, _D), jnp.float32),
        jax.ShapeDtypeStruct((_N, 1), jnp.float32),
    ],
)


def _mid_body(s_ref, g_ref, dinv_ref, w_ref, b_ref, gam_ref, bet_ref, out_ref):
    acc = s_ref[0] + s_ref[1] + g_ref[...]
    conv = dinv_ref[...] * acc + b_ref[...]
    h = jnp.maximum(conv * (gam_ref[...] * _BN_RSQRT) + bet_ref[...], 0.0)
    out_ref[...] = dinv_ref[...] * jnp.dot(
        h, w_ref[...], preferred_element_type=jnp.float32
    )


_mid = pl.pallas_call(
    _mid_body,
    grid=(_N // _BLK,),
    in_specs=[
        pl.BlockSpec((2, _BLK, _D), lambda i: (0, i, 0)),
        pl.BlockSpec((_BLK, _D), lambda i: (i, 0)),
        pl.BlockSpec((_BLK, 1), lambda i: (i, 0)),
        pl.BlockSpec((_D, _D), lambda i: (0, 0)),
        pl.BlockSpec((1, _D), lambda i: (0, 0)),
        pl.BlockSpec((1, _D), lambda i: (0, 0)),
        pl.BlockSpec((1, _D), lambda i: (0, 0)),
    ],
    out_specs=pl.BlockSpec((_BLK, _D), lambda i: (i, 0)),
    out_shape=jax.ShapeDtypeStruct((_N, _D), jnp.float32),
)


def _fin_body(s_ref, g_ref, dinv_ref, b_ref, out_ref):
    conv = dinv_ref[...] * (s_ref[0] + s_ref[1] + g_ref[...]) + b_ref[...]
    m = jnp.max(conv, axis=-1, keepdims=True)
    lse = jnp.log(jnp.sum(jnp.exp(conv - m), axis=-1, keepdims=True)) + m
    out_ref[...] = conv - lse


_fin = pl.pallas_call(
    _fin_body,
    grid=(_N // _BLK,),
    in_specs=[
        pl.BlockSpec((2, _BLK, _D), lambda i: (0, i, 0)),
        pl.BlockSpec((_BLK, _D), lambda i: (i, 0)),
        pl.BlockSpec((_BLK, 1), lambda i: (i, 0)),
        pl.BlockSpec((1, _D), lambda i: (0, 0)),
    ],
    out_specs=pl.BlockSpec((_BLK, _D), lambda i: (i, 0)),
    out_shape=jax.ShapeDtypeStruct((_N, _D), jnp.float32),
)


def kernel(x, edge_index, W, b, bn_gamma, bn_beta):
    e = edge_index.shape[1]
    nchunk = -(-e // (_NW * _CHUNK))
    nchunk = -(-nchunk // 8) * 8
    epad = _NW * nchunk * _CHUNK
    pad = epad - e
    pad_ar = jnp.arange(pad, dtype=jnp.int32)
    src = jnp.concatenate([edge_index[0], pad_ar % _N])
    dst = jnp.concatenate([edge_index[1], _N + pad_ar % (_NPAD - _N)])
    src = src.reshape(nchunk, _NW, _CHUNK).swapaxes(0, 1)
    dst = dst.reshape(nchunk, _NW, _CHUNK).swapaxes(0, 1)

    propagate = _make_propagate(nchunk)
    z = _mm(x, W[0])
    deg = _make_degree(nchunk)(dst)

    g, dinv = _pre(z, deg)
    for i in range(4):
        s = propagate(g, src, dst)
        g = _mid(
            s,
            g,
            dinv,
            W[i + 1],
            b[i].reshape(1, _D),
            bn_gamma[i].reshape(1, _D),
            bn_beta[i].reshape(1, _D),
        )
    s = propagate(g, src, dst)
    return _fin(s, g, dinv, b[4].reshape(1, _D))
